# Initial kernel scaffold; baseline (speedup 1.0000x reference)
import jax
import jax.numpy as jnp
from jax.experimental import pallas as pl


def kernel(output, target):
    raise NotImplementedError("write your pallas kernel here")


# speedup vs baseline: 1.3977x; 1.3977x over previous
import jax
import jax.numpy as jnp
from jax.experimental import pallas as pl
from jax.experimental.pallas import tpu as pltpu

_B = 128
_N = 100000
_BLK = 8192
_NBLK = (_N + _BLK - 1) // _BLK
_NEG = -3.0e38
_THR_BASE = 103.68000000000001


def _npc_body(tgt_ref, x_ref, out_ref, maxe_ref, m_ref, s_ref, picked_ref):
    i = pl.program_id(0)

    @pl.when(i == 0)
    def _init():
        maxe_ref[...] = jnp.full((_B, 1), _NEG, jnp.float32)
        m_ref[...] = jnp.full((_B, 1), _NEG, jnp.float32)
        s_ref[...] = jnp.zeros((_B, 1), jnp.float32)
        picked_ref[...] = jnp.zeros((_B, 1), jnp.float32)

    x = x_ref[...]
    col = jax.lax.broadcasted_iota(jnp.int32, (_B, _BLK), 1) + i * _BLK
    valid = col < _N
    is_tgt = col == tgt_ref[...]

    picked_ref[...] += jnp.sum(jnp.where(is_tgt, x, 0.0), axis=1, keepdims=True)

    x_all = jnp.where(valid, x, _NEG)
    x_excl = jnp.where(is_tgt, _NEG, x_all)

    maxe_ref[...] = jnp.maximum(
        maxe_ref[...], jnp.max(x_excl, axis=1, keepdims=True)
    )

    m_old = m_ref[...]
    m_new = jnp.maximum(m_old, jnp.max(x_all, axis=1, keepdims=True))
    e = jnp.where(valid, jnp.exp(x_all - m_new), 0.0)
    s_ref[...] = s_ref[...] * jnp.exp(m_old - m_new) + jnp.sum(
        e, axis=1, keepdims=True
    )
    m_ref[...] = m_new

    @pl.when(i == _NBLK - 1)
    def _epilogue():
        picked = picked_ref[...]
        margin = picked - maxe_ref[...]
        lse = m_ref[...] + jnp.log(s_ref[...])
        neg_count = jnp.sum((margin < 0).astype(jnp.float32))
        thr = jnp.floor(jnp.float32(_THR_BASE) + jnp.float32(0.9) * neg_count)
        shl = jnp.where(margin >= 0, 1.0 - margin, 1.0 - picked + lse)
        l = jnp.maximum(shl, 0.0)

        row_i = jax.lax.broadcasted_iota(jnp.int32, (_B, _B), 0)
        col_j = jax.lax.broadcasted_iota(jnp.int32, (_B, _B), 1)
        lt = jnp.sum(jnp.where(row_i == col_j, l, 0.0), axis=0, keepdims=True)
        before = (l < lt) | ((l == lt) & (row_i < col_j))
        rank = jnp.sum(before.astype(jnp.int32), axis=0, keepdims=True)
        Lk = jnp.sum(jnp.where(rank <= row_i, lt, 0.0), axis=1, keepdims=True)
        sorted_k = jnp.sum(
            jnp.where(rank == row_i, lt, 0.0), axis=1, keepdims=True
        )
        k_pos = jax.lax.broadcasted_iota(jnp.int32, (_B, 1), 0).astype(
            jnp.float32
        )
        cond = Lk <= thr + 1.0 - k_pos
        npcl1 = jnp.sum(jnp.where(cond, sorted_k, 0.0))
        npcl2 = thr - jnp.sum(cond.astype(jnp.float32))
        out_ref[...] = jnp.where(npcl1 < npcl2, npcl2, npcl1).reshape(1, 1)


def kernel(output, target):
    tgt = target.astype(jnp.int32).reshape(_B, 1)
    out = pl.pallas_call(
        _npc_body,
        grid=(_NBLK,),
        in_specs=[
            pl.BlockSpec((_B, 1), lambda i: (0, 0)),
            pl.BlockSpec((_B, _BLK), lambda i: (0, i)),
        ],
        out_specs=pl.BlockSpec((1, 1), lambda i: (0, 0)),
        out_shape=jax.ShapeDtypeStruct((1, 1), jnp.float32),
        scratch_shapes=[
            pltpu.VMEM((_B, 1), jnp.float32),
            pltpu.VMEM((_B, 1), jnp.float32),
            pltpu.VMEM((_B, 1), jnp.float32),
            pltpu.VMEM((_B, 1), jnp.float32),
        ],
        compiler_params=pltpu.CompilerParams(
            dimension_semantics=("arbitrary",),
        ),
    )(tgt, output)
    return out[0, 0]


# speedup vs baseline: 1.5086x; 1.0794x over previous
import jax
import jax.numpy as jnp
from jax.experimental import pallas as pl
from jax.experimental.pallas import tpu as pltpu

_B = 128
_N = 100000
_BLK = 16384
_NBLK = (_N + _BLK - 1) // _BLK
_TAIL = _N - (_NBLK - 1) * _BLK
_LOG2E = 1.4426950408889634
_THR_BASE = 103.68000000000001


def _npc_body(tgt_ref, x_ref, out_ref, m_ref, s_ref, picked_ref):
    i = pl.program_id(0)

    @pl.when(i == 0)
    def _init():
        m_ref[...] = jnp.full((_B, 1), -jnp.inf, jnp.float32)
        s_ref[...] = jnp.zeros((_B, 1), jnp.float32)
        picked_ref[...] = jnp.zeros((_B, 1), jnp.float32)

    lane = jax.lax.broadcasted_iota(jnp.int32, (_B, _BLK), 1)
    is_tgt = lane == tgt_ref[...] - i * _BLK
    x = x_ref[...]

    def accumulate(x_excl, px):
        picked_ref[...] += jnp.sum(px, axis=1, keepdims=True)
        m_old = m_ref[...]
        m_new = jnp.maximum(m_old, jnp.max(x_excl, axis=1, keepdims=True))
        mhat = m_new * _LOG2E
        e = jnp.exp2(x_excl * _LOG2E - mhat)
        s_ref[...] = s_ref[...] * jnp.exp2(m_old * _LOG2E - mhat) + jnp.sum(
            e, axis=1, keepdims=True
        )
        m_ref[...] = m_new

    @pl.when(i < _NBLK - 1)
    def _main():
        accumulate(
            jnp.where(is_tgt, -jnp.inf, x), jnp.where(is_tgt, x, 0.0)
        )

    @pl.when(i == _NBLK - 1)
    def _tail():
        valid = lane < _TAIL
        accumulate(
            jnp.where(valid & ~is_tgt, x, -jnp.inf),
            jnp.where(valid & is_tgt, x, 0.0),
        )

        picked = picked_ref[...]
        margin = picked - m_ref[...]
        m_all = jnp.maximum(m_ref[...], picked)
        lse = m_all + jnp.log(
            s_ref[...] * jnp.exp(m_ref[...] - m_all)
            + jnp.exp(picked - m_all)
        )
        neg_count = jnp.sum((margin < 0).astype(jnp.float32))
        thr = jnp.floor(jnp.float32(_THR_BASE) + jnp.float32(0.9) * neg_count)
        shl = jnp.where(margin >= 0, 1.0 - margin, 1.0 - picked + lse)
        l = jnp.maximum(shl, 0.0)

        row_i = jax.lax.broadcasted_iota(jnp.int32, (_B, _B), 0)
        col_j = jax.lax.broadcasted_iota(jnp.int32, (_B, _B), 1)
        lt = jnp.sum(jnp.where(row_i == col_j, l, 0.0), axis=0, keepdims=True)
        before = (l < lt) | ((l == lt) & (row_i < col_j))
        rank = jnp.sum(before.astype(jnp.int32), axis=0, keepdims=True)
        Lk = jnp.sum(jnp.where(rank <= row_i, lt, 0.0), axis=1, keepdims=True)
        sorted_k = jnp.sum(
            jnp.where(rank == row_i, lt, 0.0), axis=1, keepdims=True
        )
        k_pos = jax.lax.broadcasted_iota(jnp.int32, (_B, 1), 0).astype(
            jnp.float32
        )
        cond = Lk <= thr + 1.0 - k_pos
        npcl1 = jnp.sum(jnp.where(cond, sorted_k, 0.0))
        npcl2 = thr - jnp.sum(cond.astype(jnp.float32))
        out_ref[...] = jnp.where(npcl1 < npcl2, npcl2, npcl1).reshape(1, 1)


def kernel(output, target):
    tgt = target.astype(jnp.int32).reshape(_B, 1)
    out = pl.pallas_call(
        _npc_body,
        grid=(_NBLK,),
        in_specs=[
            pl.BlockSpec((_B, 1), lambda i: (0, 0)),
            pl.BlockSpec((_B, _BLK), lambda i: (0, i)),
        ],
        out_specs=pl.BlockSpec((1, 1), lambda i: (0, 0)),
        out_shape=jax.ShapeDtypeStruct((1, 1), jnp.float32),
        scratch_shapes=[
            pltpu.VMEM((_B, 1), jnp.float32),
            pltpu.VMEM((_B, 1), jnp.float32),
            pltpu.VMEM((_B, 1), jnp.float32),
        ],
        compiler_params=pltpu.CompilerParams(
            dimension_semantics=("arbitrary",),
        ),
    )(tgt, output)
    return out[0, 0]
